# baseline (speedup 1.0000x reference)
import functools

import jax
import jax.numpy as jnp
from jax import lax
from jax.experimental import pallas as pl
from jax.experimental.pallas import tpu as pltpu
from jax.experimental.pallas import tpu_sc as plsc

_NUM_TOKENS = 32
_DIM = 768
_BATCH = 16384
_LANES = 16

_INFO = plsc.get_sparse_core_info()
_NC = _INFO.num_cores
_NS = _INFO.num_subcores
_NW = _NC * _NS
_BPW = _BATCH // _NW
_CHUNK = 64
_NCHUNKS = _BPW // _CHUNK
_GPC = _CHUNK // _LANES
_CSTEPS = _DIM // _LANES


def _body(tokens_hbm, idx_hbm, out_hbm, table_v, idx_v, rows_v, st, s0, s1):
    wid = lax.axis_index("s") * _NC + lax.axis_index("c")
    base = wid * _BPW

    tcopy = pltpu.async_copy(tokens_hbm, table_v, st)
    icopy = pltpu.async_copy(idx_hbm.at[pl.ds(base, _BPW)], idx_v, s0)
    icopy.wait()
    tcopy.wait()

    def chunk_body(c, _):
        parity = lax.rem(c, 2)
        buf_base = parity * _CHUNK

        @pl.when(jnp.logical_and(c >= 2, parity == 0))
        def _():
            pltpu.make_async_copy(
                rows_v.at[pl.ds(0, _CHUNK)],
                out_hbm.at[pl.ds(0, _CHUNK)], s0).wait()

        @pl.when(jnp.logical_and(c >= 2, parity == 1))
        def _():
            pltpu.make_async_copy(
                rows_v.at[pl.ds(0, _CHUNK)],
                out_hbm.at[pl.ds(0, _CHUNK)], s1).wait()

        @plsc.parallel_loop(0, _GPC)
        def _fill(q):
            g = c * _GPC + q
            idxv = idx_v[pl.ds(g * _LANES, _LANES)]
            for r in range(_LANES):
                tok = idxv[r]
                buf_row = buf_base + q * _LANES + r
                for k in range(_CSTEPS):
                    rows_v[buf_row, pl.ds(k * _LANES, _LANES)] = (
                        table_v[tok, pl.ds(k * _LANES, _LANES)])

        @pl.when(parity == 0)
        def _():
            pltpu.async_copy(
                rows_v.at[pl.ds(0, _CHUNK)],
                out_hbm.at[pl.ds(base + c * _CHUNK, _CHUNK)], s0)

        @pl.when(parity == 1)
        def _():
            pltpu.async_copy(
                rows_v.at[pl.ds(_CHUNK, _CHUNK)],
                out_hbm.at[pl.ds(base + c * _CHUNK, _CHUNK)], s1)

        return _

    lax.fori_loop(0, _NCHUNKS, chunk_body, None)

    pltpu.make_async_copy(
        rows_v.at[pl.ds(0, _CHUNK)], out_hbm.at[pl.ds(0, _CHUNK)], s0).wait()
    pltpu.make_async_copy(
        rows_v.at[pl.ds(0, _CHUNK)], out_hbm.at[pl.ds(0, _CHUNK)], s1).wait()


_lookup = functools.partial(
    pl.kernel,
    out_type=jax.ShapeDtypeStruct((_BATCH, _DIM), jnp.float32),
    mesh=plsc.VectorSubcoreMesh(core_axis_name="c", subcore_axis_name="s"),
    scratch_types=[
        pltpu.VMEM((_NUM_TOKENS, _DIM), jnp.float32),
        pltpu.VMEM((_BPW,), jnp.int32),
        pltpu.VMEM((2 * _CHUNK, _DIM), jnp.float32),
        pltpu.SemaphoreType.DMA,
        pltpu.SemaphoreType.DMA,
        pltpu.SemaphoreType.DMA,
    ],
)(_body)


@jax.jit
def kernel(tokens, indices):
    return _lookup(tokens, indices)


# speedup vs baseline: 2.4108x; 2.4108x over previous
import functools

import jax
import jax.numpy as jnp
from jax import lax
from jax.experimental import pallas as pl
from jax.experimental.pallas import tpu as pltpu
from jax.experimental.pallas import tpu_sc as plsc

_NUM_TOKENS = 32
_DIM = 768
_BATCH = 16384
_LANES = 16

_INFO = plsc.get_sparse_core_info()
_NC = _INFO.num_cores
_NS = _INFO.num_subcores
_NW = _NC * _NS
_BPW = _BATCH // _NW
_NGROUPS = _BPW // _LANES
_LAG = 8


def _body(tokens_hbm, idx_hbm, out_hbm, table_sh, idx_v, st, si, sem):
    sid = lax.axis_index("s")
    wid = sid * _NC + lax.axis_index("c")
    base = wid * _BPW

    @pl.when(sid == 0)
    def _():
        pltpu.async_copy(tokens_hbm, table_sh, st).wait()

    icopy = pltpu.async_copy(idx_hbm.at[pl.ds(base, _BPW)], idx_v, si)
    icopy.wait()
    plsc.subcore_barrier()

    def group(g, _):
        idxv = idx_v[pl.ds(g * _LANES, _LANES)]
        for r in range(_LANES):
            tok = idxv[r]
            pltpu.async_copy(
                table_sh.at[tok], out_hbm.at[base + g * _LANES + r], sem)

        @pl.when(g >= _LAG)
        def _():
            pltpu.make_async_copy(
                table_sh.at[pl.ds(0, _LANES)],
                out_hbm.at[pl.ds(0, _LANES)], sem).wait()

        return _

    lax.fori_loop(0, _NGROUPS, group, None)

    def drain(i, _):
        pltpu.make_async_copy(
            table_sh.at[pl.ds(0, _LANES)],
            out_hbm.at[pl.ds(0, _LANES)], sem).wait()
        return _

    lax.fori_loop(0, _LAG, drain, None)


_lookup = functools.partial(
    pl.kernel,
    out_type=jax.ShapeDtypeStruct((_BATCH, _DIM), jnp.float32),
    mesh=plsc.VectorSubcoreMesh(core_axis_name="c", subcore_axis_name="s"),
    scratch_types=[
        pltpu.VMEM_SHARED((_NUM_TOKENS, _DIM), jnp.float32),
        pltpu.VMEM((_BPW,), jnp.int32),
        pltpu.SemaphoreType.DMA,
        pltpu.SemaphoreType.DMA,
        pltpu.SemaphoreType.DMA,
    ],
)(_body)


@jax.jit
def kernel(tokens, indices):
    return _lookup(tokens, indices)


# speedup vs baseline: 2.4553x; 1.0185x over previous
import functools

import jax
import jax.numpy as jnp
from jax import lax
from jax.experimental import pallas as pl
from jax.experimental.pallas import tpu as pltpu
from jax.experimental.pallas import tpu_sc as plsc

_NUM_TOKENS = 32
_DIM = 768
_BATCH = 16384
_LANES = 16

_TC_ROWS = 12288
_SC_ROWS = _BATCH - _TC_ROWS
_BLK = 512
_NBLK = _TC_ROWS // _BLK

_INFO = plsc.get_sparse_core_info()
_NC = _INFO.num_cores
_NS = _INFO.num_subcores
_NW = _NC * _NS
_BPW = _SC_ROWS // _NW
_NGROUPS = _BPW // _LANES
_LAG = 8


def _sc_body(tokens_hbm, idx_hbm, out_hbm, table_v, idx_v, st, si, sem):
    wid = lax.axis_index("s") * _NC + lax.axis_index("c")
    base = _TC_ROWS + wid * _BPW

    tcopy = pltpu.async_copy(tokens_hbm, table_v, st)
    icopy = pltpu.async_copy(idx_hbm.at[pl.ds(base, _BPW)], idx_v, si)
    icopy.wait()
    tcopy.wait()

    def group(g, _):
        idxv = idx_v[pl.ds(g * _LANES, _LANES)]
        for r in range(_LANES):
            tok = idxv[r]
            pltpu.async_copy(
                table_v.at[tok], out_hbm.at[base + g * _LANES + r], sem)

        @pl.when(g >= _LAG)
        def _():
            pltpu.make_async_copy(
                table_v.at[pl.ds(0, _LANES)],
                out_hbm.at[pl.ds(0, _LANES)], sem).wait()

        return _

    lax.fori_loop(0, _NGROUPS, group, None)

    def drain(i, _):
        pltpu.make_async_copy(
            table_v.at[pl.ds(0, _LANES)],
            out_hbm.at[pl.ds(0, _LANES)], sem).wait()
        return _

    lax.fori_loop(0, min(_LAG, _NGROUPS), drain, None)


_sc_lookup = functools.partial(
    pl.kernel,
    out_type=jax.ShapeDtypeStruct((_BATCH, _DIM), jnp.float32),
    mesh=plsc.VectorSubcoreMesh(core_axis_name="c", subcore_axis_name="s"),
    scratch_types=[
        pltpu.VMEM((_NUM_TOKENS, _DIM), jnp.float32),
        pltpu.VMEM((_BPW,), jnp.int32),
        pltpu.SemaphoreType.DMA,
        pltpu.SemaphoreType.DMA,
        pltpu.SemaphoreType.DMA,
    ],
)(_sc_body)


def _tc_body(idx_ref, tab_ref, prev_ref, out_ref):
    del prev_ref
    idx = idx_ref[0, 0, :]
    oh = (idx[:, None] == lax.broadcasted_iota(
        jnp.int32, (_BLK, _NUM_TOKENS), 1)).astype(jnp.float32)
    out_ref[...] = jnp.dot(oh, tab_ref[...],
                           preferred_element_type=jnp.float32)


@jax.jit
def kernel(tokens, indices):
    sc_out = _sc_lookup(tokens, indices)
    idx3 = indices[:_TC_ROWS].reshape(_NBLK, 1, _BLK)
    return pl.pallas_call(
        _tc_body,
        grid=(_NBLK,),
        in_specs=[
            pl.BlockSpec((1, 1, _BLK), lambda i: (i, 0, 0)),
            pl.BlockSpec((_NUM_TOKENS, _DIM), lambda i: (0, 0)),
            pl.BlockSpec(memory_space=pl.ANY),
        ],
        out_specs=pl.BlockSpec((_BLK, _DIM), lambda i: (i, 0)),
        out_shape=jax.ShapeDtypeStruct((_BATCH, _DIM), jnp.float32),
        input_output_aliases={2: 0},
    )(idx3, tokens, sc_out)


# speedup vs baseline: 3.0013x; 1.2224x over previous
import functools

import jax
import jax.numpy as jnp
from jax import lax
from jax.experimental import pallas as pl
from jax.experimental.pallas import tpu as pltpu
from jax.experimental.pallas import tpu_sc as plsc

_NUM_TOKENS = 32
_DIM = 768
_BATCH = 16384
_LANES = 16

_INFO = plsc.get_sparse_core_info()
_NC = _INFO.num_cores
_NS = _INFO.num_subcores
_NW = _NC * _NS
_BPW = _BATCH // _NW
_NGROUPS = _BPW // _LANES
_LAG = 8


def _body(tokens_hbm, idx_hbm, out_hbm, table_v, idx_v, st, si, sem):
    wid = lax.axis_index("s") * _NC + lax.axis_index("c")
    base = wid * _BPW

    tcopy = pltpu.async_copy(tokens_hbm, table_v, st)
    icopy = pltpu.async_copy(idx_hbm.at[pl.ds(base, _BPW)], idx_v, si)
    icopy.wait()
    tcopy.wait()

    def group(g, _):
        idxv = idx_v[pl.ds(g * _LANES, _LANES)]
        for r in range(_LANES):
            tok = idxv[r]
            pltpu.async_copy(
                table_v.at[tok], out_hbm.at[base + g * _LANES + r], sem)

        @pl.when(g >= _LAG)
        def _():
            pltpu.make_async_copy(
                table_v.at[pl.ds(0, _LANES)],
                out_hbm.at[pl.ds(0, _LANES)], sem).wait()

        return _

    lax.fori_loop(0, _NGROUPS, group, None)

    def drain(i, _):
        pltpu.make_async_copy(
            table_v.at[pl.ds(0, _LANES)],
            out_hbm.at[pl.ds(0, _LANES)], sem).wait()
        return _

    lax.fori_loop(0, _LAG, drain, None)


_lookup = functools.partial(
    pl.kernel,
    out_type=jax.ShapeDtypeStruct((_BATCH, _DIM), jnp.float32),
    mesh=plsc.VectorSubcoreMesh(core_axis_name="c", subcore_axis_name="s"),
    scratch_types=[
        pltpu.VMEM((_NUM_TOKENS, _DIM), jnp.float32),
        pltpu.VMEM((_BPW,), jnp.int32),
        pltpu.SemaphoreType.DMA,
        pltpu.SemaphoreType.DMA,
        pltpu.SemaphoreType.DMA,
    ],
)(_body)


@jax.jit
def kernel(tokens, indices):
    return _lookup(tokens, indices)
